# baseline (speedup 1.0000x reference)
import functools

import jax
import jax.numpy as jnp
from jax import lax
from jax.experimental import pallas as pl
from jax.experimental.pallas import tpu as pltpu
from jax.experimental.pallas import tpu_sc as plsc

B, G, T, E, H = 8, 64, 512, 128, 1024
NW = 32
PP = (B * G) // NW
TE = T * E


def _mm_body(h_ref, w_ref, o_ref):
    o_ref[...] = jnp.dot(h_ref[...], w_ref[...],
                         preferred_element_type=jnp.float32)


_tc_matmul = pl.pallas_call(
    _mm_body,
    out_shape=jax.ShapeDtypeStruct((B, E), jnp.float32),
)


def _sc_body(emb_hbm, inter_hbm, atop_hbm, alpha_hbm, cpart_hbm,
             x_v, inter_v, atop_v, beta_v, c_v):
    wid = lax.axis_index("s") * 2 + lax.axis_index("c")
    b = wid // (NW // B)
    pltpu.sync_copy(inter_hbm.at[b], inter_v)
    pltpu.sync_copy(atop_hbm.at[pl.ds(wid * PP, PP)], atop_v)

    zero16 = jnp.zeros((16,), jnp.float32)
    iota16 = lax.iota(jnp.int32, 16)
    for eb in range(E // 16):
        c_v[pl.ds(eb * 16, 16)] = zero16

    def pair_body(i, _):
        p = wid * PP + i
        pltpu.sync_copy(emb_hbm.at[pl.ds(p * TE, TE)], x_v)

        def tb_body(tb, _):
            ib = (tb * 16 + iota16) * E

            def ec_body(ec, carry):
                a0, a1, a2, a3, idx = carry
                iv16 = inter_v[pl.ds(ec * 16, 16)]
                accs = [a0, a1, a2, a3]
                for k in range(16):
                    xk = plsc.load_gather(x_v, [idx + k if k else idx])
                    wk = jnp.full((16,), iv16[k], jnp.float32)
                    accs[k % 4] = accs[k % 4] + xk * wk
                return accs[0], accs[1], accs[2], accs[3], idx + 16

            a0, a1, a2, a3, _ = lax.fori_loop(
                0, E // 16, ec_body, (zero16, zero16, zero16, zero16, ib))
            beta_v[pl.ds(tb * 16, 16)] = (a0 + a1) + (a2 + a3)
            return 0

        lax.fori_loop(0, T // 16, tb_body, 0)

        def mx_body(j, m):
            return jnp.maximum(m, beta_v[pl.ds(j * 16, 16)])

        mv = lax.fori_loop(0, T // 16, mx_body,
                           jnp.full((16,), -jnp.inf, jnp.float32))
        ms = jnp.full((16,), jnp.max(mv), jnp.float32)

        def ex_body(j, s):
            ev = jnp.exp(beta_v[pl.ds(j * 16, 16)] - ms)
            beta_v[pl.ds(j * 16, 16)] = ev
            return s + ev

        sv = lax.fori_loop(0, T // 16, ex_body, zero16)
        atop_reg = atop_v[...]
        atop_i = jnp.sum(jnp.where(iota16 == i, atop_reg, 0.0))
        scs = (jnp.full((16,), atop_i, jnp.float32)
               / jnp.full((16,), jnp.sum(sv), jnp.float32))

        def al_body(j, _):
            beta_v[pl.ds(j * 16, 16)] = beta_v[pl.ds(j * 16, 16)] * scs
            return 0

        lax.fori_loop(0, T // 16, al_body, 0)
        pltpu.sync_copy(beta_v, alpha_hbm.at[p])

        def tb2_body(tb, accs):
            av16 = beta_v[pl.ds(tb * 16, 16)]
            base = tb * (16 * E)
            accs = list(accs)
            for lane in range(16):
                av = jnp.full((16,), av16[lane], jnp.float32)
                toff = base + lane * E
                for eb in range(E // 16):
                    accs[eb] = accs[eb] + x_v[pl.ds(toff + eb * 16, 16)] * av
            return tuple(accs)

        accs = lax.fori_loop(0, T // 16, tb2_body, (zero16,) * (E // 16))
        for eb in range(E // 16):
            c_v[pl.ds(eb * 16, 16)] = c_v[pl.ds(eb * 16, 16)] + accs[eb]
        return 0

    lax.fori_loop(0, PP, pair_body, 0)
    pltpu.sync_copy(c_v, cpart_hbm.at[wid])


_sc_call = functools.partial(
    pl.kernel,
    mesh=plsc.VectorSubcoreMesh(core_axis_name="c", subcore_axis_name="s"),
    compiler_params=pltpu.CompilerParams(needs_layout_passes=False),
    out_type=(
        jax.ShapeDtypeStruct((B * G, T), jnp.float32),
        jax.ShapeDtypeStruct((NW, E), jnp.float32),
    ),
    scratch_types=[
        pltpu.VMEM((TE,), jnp.float32),
        pltpu.VMEM((E,), jnp.float32),
        pltpu.VMEM((PP,), jnp.float32),
        pltpu.VMEM((T,), jnp.float32),
        pltpu.VMEM((E,), jnp.float32),
    ],
)(_sc_body)


def kernel(decoder_hidden_state, alpha_graph_attention_top, all_embeddings, W):
    inter = _tc_matmul(decoder_hidden_state, W)
    emb_flat = all_embeddings.reshape(-1)
    atop_flat = alpha_graph_attention_top.reshape(-1)
    alpha_flat, c_part = _sc_call(emb_flat, inter, atop_flat)
    c = c_part.reshape(B, NW // B, E).sum(axis=1)
    alpha = alpha_flat.reshape(B, G, T)
    return (c, alpha)


# speedup vs baseline: 3.0377x; 3.0377x over previous
import functools

import jax
import jax.numpy as jnp
from jax import lax
from jax.experimental import pallas as pl
from jax.experimental.pallas import tpu as pltpu
from jax.experimental.pallas import tpu_sc as plsc

B, G, T, E, H = 8, 64, 512, 128, 1024
NW = 32
PP = (B * G) // NW
TE = T * E


def _mm_body(h_ref, w_ref, o_ref):
    o_ref[...] = jnp.dot(h_ref[...], w_ref[...],
                         preferred_element_type=jnp.float32)


_tc_matmul = pl.pallas_call(
    _mm_body,
    out_shape=jax.ShapeDtypeStruct((B, E), jnp.float32),
)


def _sc_body(emb_hbm, inter_hbm, atop_hbm, alpha_hbm, cpart_hbm,
             x_v, inter_v, atop_v, beta_v, c_v):
    wid = lax.axis_index("s") * 2 + lax.axis_index("c")
    b = wid // (NW // B)
    pltpu.sync_copy(inter_hbm.at[b], inter_v)
    pltpu.sync_copy(atop_hbm.at[pl.ds(wid * PP, PP)], atop_v)

    zero16 = jnp.zeros((16,), jnp.float32)
    iota16 = lax.iota(jnp.int32, 16)
    for eb in range(E // 16):
        c_v[pl.ds(eb * 16, 16)] = zero16

    ivs = [inter_v[pl.ds(eb * 16, 16)] for eb in range(E // 16)]

    def pair_body(i, _):
        p = wid * PP + i
        pltpu.sync_copy(emb_hbm.at[pl.ds(p * TE, TE)], x_v)

        def tb_body(tb, _):
            base0 = tb * (16 * E)
            betav = zero16
            for lane in range(16):
                roff = base0 + lane * E
                m = [x_v[pl.ds(roff + eb * 16, 16)] * ivs[eb]
                     for eb in range(E // 16)]
                r = ((m[0] + m[1]) + (m[2] + m[3])) + \
                    ((m[4] + m[5]) + (m[6] + m[7]))
                betav = jnp.where(iota16 == lane, jnp.sum(r), betav)
            beta_v[pl.ds(tb * 16, 16)] = betav
            return 0

        lax.fori_loop(0, T // 16, tb_body, 0)

        def mx_body(j, m):
            return jnp.maximum(m, beta_v[pl.ds(j * 16, 16)])

        mv = lax.fori_loop(0, T // 16, mx_body,
                           jnp.full((16,), -jnp.inf, jnp.float32))
        ms = jnp.full((16,), jnp.max(mv), jnp.float32)

        def ex_body(j, s):
            ev = jnp.exp(beta_v[pl.ds(j * 16, 16)] - ms)
            beta_v[pl.ds(j * 16, 16)] = ev
            return s + ev

        sv = lax.fori_loop(0, T // 16, ex_body, zero16)
        atop_reg = atop_v[...]
        atop_i = jnp.sum(jnp.where(iota16 == i, atop_reg, 0.0))
        scs = (jnp.full((16,), atop_i, jnp.float32)
               / jnp.full((16,), jnp.sum(sv), jnp.float32))

        def al_body(j, _):
            beta_v[pl.ds(j * 16, 16)] = beta_v[pl.ds(j * 16, 16)] * scs
            return 0

        lax.fori_loop(0, T // 16, al_body, 0)
        pltpu.sync_copy(beta_v, alpha_hbm.at[p])

        def tb2_body(tb, accs):
            av16 = beta_v[pl.ds(tb * 16, 16)]
            base = tb * (16 * E)
            accs = list(accs)
            for lane in range(16):
                av = jnp.full((16,), av16[lane], jnp.float32)
                toff = base + lane * E
                for eb in range(E // 16):
                    accs[eb] = accs[eb] + x_v[pl.ds(toff + eb * 16, 16)] * av
            return tuple(accs)

        accs = lax.fori_loop(0, T // 16, tb2_body, (zero16,) * (E // 16))
        for eb in range(E // 16):
            c_v[pl.ds(eb * 16, 16)] = c_v[pl.ds(eb * 16, 16)] + accs[eb]
        return 0

    lax.fori_loop(0, PP, pair_body, 0)
    pltpu.sync_copy(c_v, cpart_hbm.at[wid])


_sc_call = functools.partial(
    pl.kernel,
    mesh=plsc.VectorSubcoreMesh(core_axis_name="c", subcore_axis_name="s"),
    compiler_params=pltpu.CompilerParams(needs_layout_passes=False),
    out_type=(
        jax.ShapeDtypeStruct((B * G, T), jnp.float32),
        jax.ShapeDtypeStruct((NW, E), jnp.float32),
    ),
    scratch_types=[
        pltpu.VMEM((TE,), jnp.float32),
        pltpu.VMEM((E,), jnp.float32),
        pltpu.VMEM((PP,), jnp.float32),
        pltpu.VMEM((T,), jnp.float32),
        pltpu.VMEM((E,), jnp.float32),
    ],
)(_sc_body)


def kernel(decoder_hidden_state, alpha_graph_attention_top, all_embeddings, W):
    inter = _tc_matmul(decoder_hidden_state, W)
    emb_flat = all_embeddings.reshape(-1)
    atop_flat = alpha_graph_attention_top.reshape(-1)
    alpha_flat, c_part = _sc_call(emb_flat, inter, atop_flat)
    c = c_part.reshape(B, NW // B, E).sum(axis=1)
    alpha = alpha_flat.reshape(B, G, T)
    return (c, alpha)


# speedup vs baseline: 3.6215x; 1.1922x over previous
import functools

import jax
import jax.numpy as jnp
from jax import lax
from jax.experimental import pallas as pl
from jax.experimental.pallas import tpu as pltpu
from jax.experimental.pallas import tpu_sc as plsc

B, G, T, E, H = 8, 64, 512, 128, 1024
NW = 32
PP = (B * G) // NW
TE = T * E
HT = T // 2
HTE = HT * E
EB = E // 16


def _mm_body(h_ref, w_ref, o_ref):
    o_ref[...] = jnp.dot(h_ref[...], w_ref[...],
                         preferred_element_type=jnp.float32)


_tc_matmul = pl.pallas_call(
    _mm_body,
    out_shape=jax.ShapeDtypeStruct((B, E), jnp.float32),
)


def _sc_body(emb_hbm, inter_hbm, atop_hbm, alpha_hbm, cpart_hbm,
             h0, h1, h2, inter_v, atop_v, beta_v, c_v,
             s0, s1, s2, s_alpha):
    wid = lax.axis_index("s") * 2 + lax.axis_index("c")
    b = wid // (NW // B)
    pltpu.sync_copy(inter_hbm.at[b], inter_v)
    pltpu.sync_copy(atop_hbm.at[pl.ds(wid * PP, PP)], atop_v)

    zero16 = jnp.zeros((16,), jnp.float32)
    iota16 = lax.iota(jnp.int32, 16)
    for eb in range(EB):
        c_v[pl.ds(eb * 16, 16)] = zero16
    ivs = [inter_v[pl.ds(eb * 16, 16)] for eb in range(EB)]
    atop_reg = atop_v[...]

    def dma_start(p, half, buf, sem):
        pltpu.async_copy(
            emb_hbm.at[pl.ds(p * TE + half * HTE, HTE)], buf, sem)

    def dma_wait(p, half, buf, sem):
        pltpu.make_async_copy(
            emb_hbm.at[pl.ds(p * TE + half * HTE, HTE)], buf, sem).wait()

    def stage1_half(buf, beta_off):
        def tb_body(tb, _):
            base0 = tb * (16 * E)
            betav = zero16
            for lane in range(16):
                roff = base0 + lane * E
                m = [buf[pl.ds(roff + eb * 16, 16)] * ivs[eb]
                     for eb in range(EB)]
                r = ((m[0] + m[1]) + (m[2] + m[3])) + \
                    ((m[4] + m[5]) + (m[6] + m[7]))
                betav = jnp.where(iota16 == lane, jnp.sum(r), betav)
            beta_v[pl.ds(beta_off + tb * 16, 16)] = betav
            return 0

        lax.fori_loop(0, HT // 16, tb_body, 0)

    def softmax_scale(i):
        def mx_body(j, mv):
            return jnp.maximum(mv, beta_v[pl.ds(j * 16, 16)])

        mv = lax.fori_loop(0, T // 16, mx_body,
                           jnp.full((16,), -jnp.inf, jnp.float32))
        ms = jnp.full((16,), jnp.max(mv), jnp.float32)

        def ex_body(j, s):
            ev = jnp.exp(beta_v[pl.ds(j * 16, 16)] - ms)
            beta_v[pl.ds(j * 16, 16)] = ev
            return s + ev

        sv = lax.fori_loop(0, T // 16, ex_body, zero16)
        atop_i = jnp.sum(jnp.where(iota16 == i, atop_reg, 0.0))
        scs = (jnp.full((16,), atop_i, jnp.float32)
               / jnp.full((16,), jnp.sum(sv), jnp.float32))

        def al_body(j, _):
            beta_v[pl.ds(j * 16, 16)] = beta_v[pl.ds(j * 16, 16)] * scs
            return 0

        lax.fori_loop(0, T // 16, al_body, 0)

    def stage2_half(buf, beta_off, accs):
        def tb2_body(tb, accs):
            av16 = beta_v[pl.ds(beta_off + tb * 16, 16)]
            base = tb * (16 * E)
            accs = list(accs)
            for lane in range(16):
                av = jnp.full((16,), av16[lane], jnp.float32)
                toff = base + lane * E
                for eb in range(EB):
                    accs[eb] = accs[eb] + buf[pl.ds(toff + eb * 16, 16)] * av
            return tuple(accs)

        return lax.fori_loop(0, HT // 16, tb2_body, accs)

    def do_pair(i, lo, hi, nxt, s_lo, s_hi, s_nxt):
        p = wid * PP + i
        dma_start(p, 1, hi, s_hi)
        dma_wait(p, 0, lo, s_lo)
        stage1_half(lo, 0)
        dma_wait(p, 1, hi, s_hi)
        stage1_half(hi, HT)
        softmax_scale(i)
        pltpu.async_copy(beta_v, alpha_hbm.at[p], s_alpha)
        pn = jnp.minimum(p + 1, B * G - 1)
        dma_start(pn, 0, nxt, s_nxt)
        accs = stage2_half(lo, 0, (zero16,) * EB)
        accs = stage2_half(hi, HT, accs)
        for eb in range(EB):
            c_v[pl.ds(eb * 16, 16)] = c_v[pl.ds(eb * 16, 16)] + accs[eb]
        pltpu.make_async_copy(beta_v, alpha_hbm.at[p], s_alpha).wait()

    dma_start(wid * PP, 0, h0, s0)
    do_pair(0, h0, h1, h2, s0, s1, s2)

    def k_body(k, _):
        i1 = 1 + 3 * k
        do_pair(i1, h2, h0, h1, s2, s0, s1)
        do_pair(i1 + 1, h1, h2, h0, s1, s2, s0)
        do_pair(i1 + 2, h0, h1, h2, s0, s1, s2)
        return 0

    lax.fori_loop(0, (PP - 1) // 3, k_body, 0)
    pltpu.make_async_copy(
        emb_hbm.at[pl.ds(0, HTE)], h2, s2).wait()
    pltpu.sync_copy(c_v, cpart_hbm.at[wid])


_sc_call = functools.partial(
    pl.kernel,
    mesh=plsc.VectorSubcoreMesh(core_axis_name="c", subcore_axis_name="s"),
    compiler_params=pltpu.CompilerParams(needs_layout_passes=False),
    out_type=(
        jax.ShapeDtypeStruct((B * G, T), jnp.float32),
        jax.ShapeDtypeStruct((NW, E), jnp.float32),
    ),
    scratch_types=[
        pltpu.VMEM((HTE,), jnp.float32),
        pltpu.VMEM((HTE,), jnp.float32),
        pltpu.VMEM((HTE,), jnp.float32),
        pltpu.VMEM((E,), jnp.float32),
        pltpu.VMEM((PP,), jnp.float32),
        pltpu.VMEM((T,), jnp.float32),
        pltpu.VMEM((E,), jnp.float32),
        pltpu.SemaphoreType.DMA,
        pltpu.SemaphoreType.DMA,
        pltpu.SemaphoreType.DMA,
        pltpu.SemaphoreType.DMA,
    ],
)(_sc_body)


def kernel(decoder_hidden_state, alpha_graph_attention_top, all_embeddings, W):
    inter = _tc_matmul(decoder_hidden_state, W)
    emb_flat = all_embeddings.reshape(-1)
    atop_flat = alpha_graph_attention_top.reshape(-1)
    alpha_flat, c_part = _sc_call(emb_flat, inter, atop_flat)
    c = c_part.reshape(B, NW // B, E).sum(axis=1)
    alpha = alpha_flat.reshape(B, G, T)
    return (c, alpha)
